# baseline (speedup 1.0000x reference)
import functools

import jax
import jax.numpy as jnp
from jax import lax
from jax.experimental import pallas as pl
from jax.experimental.pallas import tpu as pltpu
from jax.experimental.pallas import tpu_sc as plsc

N_NODES = 10000
N_EDGES = 640000
N_IN = 512
N_OUT = 256
ITERATIONS = 150
LEAK = 0.01

NC = 2
NS = 16
NW = NC * NS
L = 16
NPAD = 10240
SHARE = NPAD // NS
ROWS128 = NPAD // 128
RPS = ROWS128 // NS
EPT = N_EDGES // NW
GROUPS = EPT // L
UNROLL = 8
COLBITS = 14
COLMASK = (1 << COLBITS) - 1


def _act(z):
    a = jnp.where(z < 0.0, z * LEAK, z)
    sat = a > 0.5
    xs = jnp.where(sat, a, 1.0)
    return jnp.where(sat, 1.0 - 0.25 / xs, a)


def _sc_body(rc_hbm, w_hbm, bias_hbm, x_hbm, win_hbm, iidx_hbm, oidx_hbm,
             wout_hbm, out_hbm, part_hbm, rc_v, w_v, y_v, acc_v, bin_v,
             psum_v, pprt_v, zb_v, idx80_v, iidx_v, x_v, win_v, oidx_v,
             wout_v, oval_v, sh_sum, sh_y, xsem, dsem):
    cid = lax.axis_index("c")
    sid = lax.axis_index("s")
    eid = cid * NS + sid
    lo = sid * SHARE
    elo = eid * EPT

    pltpu.sync_copy(rc_hbm.at[pl.ds(elo, EPT)], rc_v)
    pltpu.sync_copy(w_hbm.at[pl.ds(elo, EPT)], w_v)
    pltpu.sync_copy(bias_hbm.at[pl.ds(lo, SHARE)], bin_v)
    pltpu.sync_copy(iidx_hbm, iidx_v)
    pltpu.sync_copy(x_hbm, x_v)
    pltpu.sync_copy(win_hbm, win_v)

    def inproj(g, c):
        ii = iidx_v[pl.ds(g * L, L)]
        v = x_v[pl.ds(g * L, L)] * win_v[pl.ds(g * L, L)]
        m = (ii >= lo) & (ii < lo + SHARE)
        iloc = jnp.where(m, ii - lo, 0)
        plsc.addupdate_scatter(bin_v, [iloc], jnp.where(m, v, 0.0), mask=m)
        return c

    lax.fori_loop(0, N_IN // L, inproj, 0)

    zeros = jnp.zeros((L,), jnp.float32)
    lanes = lax.iota(jnp.int32, L)

    def init_bufs(i, c):
        y_v[pl.ds(i * L, L)] = zeros
        r = i >> 3
        acc_v[r, pl.ds((i & 7) * L, L)] = zeros
        return c

    lax.fori_loop(0, NPAD // L, init_bufs, 0)

    def init_zb(i, c):
        zb_v[i >> 3, pl.ds((i & 7) * L, L)] = zeros
        return c

    lax.fori_loop(0, RPS * 8, init_zb, 0)

    def init_idx(g, c):
        j = lanes + g * L
        k = j + (ROWS128 - RPS) * 0 - sid * RPS
        idx80_v[pl.ds(g * L, L)] = jnp.where(k < 0, k + ROWS128, k)
        return c

    lax.fori_loop(0, ROWS128 // L, init_idx, 0)

    pltpu.sync_copy(zb_v, sh_sum.at[pl.ds(sid * RPS, RPS)])

    def iteration(it, c):
        @plsc.parallel_loop(0, GROUPS, unroll=UNROLL)
        def egrp(g):
            base = g * L
            rc = rc_v[pl.ds(base, L)]
            col = rc & COLMASK
            row = rc >> COLBITS
            yv = plsc.load_gather(y_v, [col])
            wv = w_v[pl.ds(base, L)]
            ar = (row >> 7) + sid * RPS
            ar = jnp.where(ar >= ROWS128, ar - ROWS128, ar)
            plsc.addupdate_scatter(acc_v, [ar, row & 127], yv * wv)

        pltpu.sync_copy(acc_v, sh_sum.at[idx80_v], add=True)
        plsc.subcore_barrier()
        pltpu.sync_copy(sh_sum.at[pl.ds(sid * RPS, RPS)], psum_v)

        par = lax.rem(it, 2)
        pltpu.sync_copy(psum_v, part_hbm.at[par, cid, sid])
        pltpu.semaphore_signal(xsem, 1, core_index=1 - cid)
        pltpu.sync_copy(zb_v, sh_sum.at[pl.ds(sid * RPS, RPS)])
        pl.semaphore_wait(xsem, 1)
        pltpu.sync_copy(part_hbm.at[par, 1 - cid, sid], pprt_v)

        @plsc.parallel_loop(0, SHARE // L, unroll=4)
        def red2(g):
            r = g >> 3
            cc = (g & 7) * L
            z = (psum_v[r, pl.ds(cc, L)] + pprt_v[r, pl.ds(cc, L)]
                 + bin_v[pl.ds(g * L, L)])
            y_v[pl.ds(lo + g * L, L)] = _act(z)

        pltpu.sync_copy(y_v.at[pl.ds(lo, SHARE)], sh_y.at[pl.ds(lo, SHARE)])
        plsc.subcore_barrier()
        desc = pltpu.async_copy(sh_y, y_v, dsem)

        @plsc.parallel_loop(0, ROWS128, unroll=8)
        def zero_acc(r):
            for u in range(8):
                acc_v[r, pl.ds(u * L, L)] = zeros

        desc.wait()
        return c

    lax.fori_loop(0, ITERATIONS, iteration, 0)

    @pl.when(cid == 0)
    def _():
        pltpu.sync_copy(oidx_hbm.at[pl.ds(sid * L, L)], oidx_v)
        pltpu.sync_copy(wout_hbm.at[pl.ds(sid * L, L)], wout_v)
        oi = oidx_v[pl.ds(0, L)]
        ov = plsc.load_gather(y_v, [oi]) * wout_v[pl.ds(0, L)]
        oval_v[pl.ds(0, L)] = ov
        pltpu.sync_copy(oval_v, out_hbm.at[pl.ds(sid * L, L)])


_sc_kernel = functools.partial(
    pl.kernel,
    out_type=(jax.ShapeDtypeStruct((N_OUT,), jnp.float32),
              jax.ShapeDtypeStruct((2, NC, NS, RPS, 128), jnp.float32)),
    mesh=plsc.VectorSubcoreMesh(
        core_axis_name="c", subcore_axis_name="s", num_cores=NC),
    compiler_params=pltpu.CompilerParams(needs_layout_passes=False),
    scratch_types=[
        pltpu.VMEM((EPT,), jnp.int32),
        pltpu.VMEM((EPT,), jnp.float32),
        pltpu.VMEM((NPAD,), jnp.float32),
        pltpu.VMEM((ROWS128, 128), jnp.float32),
        pltpu.VMEM((SHARE,), jnp.float32),
        pltpu.VMEM((RPS, 128), jnp.float32),
        pltpu.VMEM((RPS, 128), jnp.float32),
        pltpu.VMEM((RPS, 128), jnp.float32),
        pltpu.VMEM((ROWS128,), jnp.int32),
        pltpu.VMEM((N_IN,), jnp.int32),
        pltpu.VMEM((N_IN,), jnp.float32),
        pltpu.VMEM((N_IN,), jnp.float32),
        pltpu.VMEM((L,), jnp.int32),
        pltpu.VMEM((L,), jnp.float32),
        pltpu.VMEM((L,), jnp.float32),
        pltpu.VMEM_SHARED((ROWS128, 128), jnp.float32),
        pltpu.VMEM_SHARED((NPAD,), jnp.float32),
        pltpu.SemaphoreType.REGULAR,
        pltpu.SemaphoreType.DMA,
    ],
)(_sc_body)


def kernel(x, w_in, w_rec, bias, w_out, rows, cols, in_idx, out_idx):
    rc = rows.astype(jnp.int32) * (1 << COLBITS) + cols.astype(jnp.int32)
    bias_pad = jnp.pad(bias.reshape(-1), (0, NPAD - N_NODES))
    out, _ = _sc_kernel(rc, w_rec, bias_pad, x.reshape(-1), w_in,
                        in_idx, out_idx, w_out)
    return out.reshape(1, N_OUT)


# speedup vs baseline: 1.0146x; 1.0146x over previous
import functools

import jax
import jax.numpy as jnp
from jax import lax
from jax.experimental import pallas as pl
from jax.experimental.pallas import tpu as pltpu
from jax.experimental.pallas import tpu_sc as plsc

N_NODES = 10000
N_EDGES = 640000
N_IN = 512
N_OUT = 256
ITERATIONS = 150
LEAK = 0.01

NC = 2
NS = 16
NW = NC * NS
L = 16
NPAD = 10240
SHARE = NPAD // NS
ROWS128 = NPAD // 128
RPS = ROWS128 // NS
EPT = N_EDGES // NW
GROUPS = EPT // L
UNROLL = 10
COLBITS = 14
COLMASK = (1 << COLBITS) - 1


def _act(z):
    a = jnp.where(z < 0.0, z * LEAK, z)
    sat = a > 0.5
    xs = jnp.where(sat, a, 1.0)
    return jnp.where(sat, 1.0 - 0.25 / xs, a)


def _sc_body(rc_hbm, w_hbm, bias_hbm, x_hbm, win_hbm, iidx_hbm, oidx_hbm,
             wout_hbm, out_hbm, part_hbm, rc_v, w_v, y_v, acc_v, bin_v,
             psum_v, pprt_v, zb_v, idx80_v, iidx_v, x_v, win_v, oidx_v,
             wout_v, oval_v, sh_sum, sh_y, xsem, dsem):
    cid = lax.axis_index("c")
    sid = lax.axis_index("s")
    eid = cid * NS + sid
    lo = sid * SHARE
    elo = eid * EPT

    pltpu.sync_copy(rc_hbm.at[pl.ds(elo, EPT)], rc_v)
    pltpu.sync_copy(w_hbm.at[pl.ds(elo, EPT)], w_v)
    pltpu.sync_copy(bias_hbm.at[pl.ds(lo, SHARE)], bin_v)
    pltpu.sync_copy(iidx_hbm, iidx_v)
    pltpu.sync_copy(x_hbm, x_v)
    pltpu.sync_copy(win_hbm, win_v)

    def inproj(g, c):
        ii = iidx_v[pl.ds(g * L, L)]
        v = x_v[pl.ds(g * L, L)] * win_v[pl.ds(g * L, L)]
        m = (ii >= lo) & (ii < lo + SHARE)
        iloc = jnp.where(m, ii - lo, 0)
        plsc.addupdate_scatter(bin_v, [iloc], jnp.where(m, v, 0.0), mask=m)
        return c

    lax.fori_loop(0, N_IN // L, inproj, 0)

    zeros = jnp.zeros((L,), jnp.float32)
    lanes = lax.iota(jnp.int32, L)

    def init_bufs(i, c):
        y_v[pl.ds(i * L, L)] = zeros
        r = i >> 3
        acc_v[r, pl.ds((i & 7) * L, L)] = zeros
        return c

    lax.fori_loop(0, NPAD // L, init_bufs, 0)

    def init_zb(i, c):
        zb_v[i >> 3, pl.ds((i & 7) * L, L)] = zeros
        return c

    lax.fori_loop(0, RPS * 8, init_zb, 0)

    def init_idx(g, c):
        idx80_v[pl.ds(g * L, L)] = lanes + g * L
        return c

    lax.fori_loop(0, ROWS128 // L, init_idx, 0)

    pltpu.sync_copy(zb_v, sh_sum.at[pl.ds(sid * RPS, RPS)])

    def iteration(it, c):
        @plsc.parallel_loop(0, GROUPS, unroll=UNROLL)
        def egrp(g):
            base = g * L
            rc = rc_v[pl.ds(base, L)]
            col = rc & COLMASK
            row = rc >> COLBITS
            yv = plsc.load_gather(y_v, [col])
            wv = w_v[pl.ds(base, L)]
            plsc.addupdate_scatter(acc_v, [row >> 7, row & 127], yv * wv)

        pltpu.sync_copy(acc_v, sh_sum.at[idx80_v], add=True)
        plsc.subcore_barrier()
        pltpu.sync_copy(sh_sum.at[pl.ds(sid * RPS, RPS)], psum_v)

        par = lax.rem(it, 2)
        pltpu.sync_copy(psum_v, part_hbm.at[par, cid, sid])
        pltpu.semaphore_signal(xsem, 1, core_index=1 - cid)
        pltpu.sync_copy(zb_v, sh_sum.at[pl.ds(sid * RPS, RPS)])
        pl.semaphore_wait(xsem, 1)
        pltpu.sync_copy(part_hbm.at[par, 1 - cid, sid], pprt_v)

        @plsc.parallel_loop(0, SHARE // L, unroll=4)
        def red2(g):
            r = g >> 3
            cc = (g & 7) * L
            z = (psum_v[r, pl.ds(cc, L)] + pprt_v[r, pl.ds(cc, L)]
                 + bin_v[pl.ds(g * L, L)])
            y_v[pl.ds(lo + g * L, L)] = _act(z)

        pltpu.sync_copy(y_v.at[pl.ds(lo, SHARE)], sh_y.at[pl.ds(lo, SHARE)])
        plsc.subcore_barrier()
        desc = pltpu.async_copy(sh_y, y_v, dsem)

        @plsc.parallel_loop(0, ROWS128, unroll=8)
        def zero_acc(r):
            for u in range(8):
                acc_v[r, pl.ds(u * L, L)] = zeros

        desc.wait()
        return c

    lax.fori_loop(0, ITERATIONS, iteration, 0)

    @pl.when(cid == 0)
    def _():
        pltpu.sync_copy(oidx_hbm.at[pl.ds(sid * L, L)], oidx_v)
        pltpu.sync_copy(wout_hbm.at[pl.ds(sid * L, L)], wout_v)
        oi = oidx_v[pl.ds(0, L)]
        ov = plsc.load_gather(y_v, [oi]) * wout_v[pl.ds(0, L)]
        oval_v[pl.ds(0, L)] = ov
        pltpu.sync_copy(oval_v, out_hbm.at[pl.ds(sid * L, L)])


_sc_kernel = functools.partial(
    pl.kernel,
    out_type=(jax.ShapeDtypeStruct((N_OUT,), jnp.float32),
              jax.ShapeDtypeStruct((2, NC, NS, RPS, 128), jnp.float32)),
    mesh=plsc.VectorSubcoreMesh(
        core_axis_name="c", subcore_axis_name="s", num_cores=NC),
    compiler_params=pltpu.CompilerParams(needs_layout_passes=False),
    scratch_types=[
        pltpu.VMEM((EPT,), jnp.int32),
        pltpu.VMEM((EPT,), jnp.float32),
        pltpu.VMEM((NPAD,), jnp.float32),
        pltpu.VMEM((ROWS128, 128), jnp.float32),
        pltpu.VMEM((SHARE,), jnp.float32),
        pltpu.VMEM((RPS, 128), jnp.float32),
        pltpu.VMEM((RPS, 128), jnp.float32),
        pltpu.VMEM((RPS, 128), jnp.float32),
        pltpu.VMEM((ROWS128,), jnp.int32),
        pltpu.VMEM((N_IN,), jnp.int32),
        pltpu.VMEM((N_IN,), jnp.float32),
        pltpu.VMEM((N_IN,), jnp.float32),
        pltpu.VMEM((L,), jnp.int32),
        pltpu.VMEM((L,), jnp.float32),
        pltpu.VMEM((L,), jnp.float32),
        pltpu.VMEM_SHARED((ROWS128, 128), jnp.float32),
        pltpu.VMEM_SHARED((NPAD,), jnp.float32),
        pltpu.SemaphoreType.REGULAR,
        pltpu.SemaphoreType.DMA,
    ],
)(_sc_body)


def kernel(x, w_in, w_rec, bias, w_out, rows, cols, in_idx, out_idx):
    rc = rows.astype(jnp.int32) * (1 << COLBITS) + cols.astype(jnp.int32)
    bias_pad = jnp.pad(bias.reshape(-1), (0, NPAD - N_NODES))
    out, _ = _sc_kernel(rc, w_rec, bias_pad, x.reshape(-1), w_in,
                        in_idx, out_idx, w_out)
    return out.reshape(1, N_OUT)


# speedup vs baseline: 1.0175x; 1.0028x over previous
import functools

import jax
import jax.numpy as jnp
from jax import lax
from jax.experimental import pallas as pl
from jax.experimental.pallas import tpu as pltpu
from jax.experimental.pallas import tpu_sc as plsc

N_NODES = 10000
N_EDGES = 640000
N_IN = 512
N_OUT = 256
ITERATIONS = 150
LEAK = 0.01

NC = 2
NS = 16
NW = NC * NS
L = 16
NPAD = 10240
SHARE = NPAD // NS
ROWS128 = NPAD // 128
RPS = ROWS128 // NS
EPT = N_EDGES // NW
GROUPS = EPT // L
UNROLL = 8
COLBITS = 14
COLMASK = (1 << COLBITS) - 1


def _act(z):
    a = jnp.where(z < 0.0, z * LEAK, z)
    sat = a > 0.5
    xs = jnp.where(sat, a, 1.0)
    return jnp.where(sat, 1.0 - 0.25 / xs, a)


def _sc_body(rc_hbm, w_hbm, bias_hbm, x_hbm, win_hbm, iidx_hbm, oidx_hbm,
             wout_hbm, out_hbm, part_hbm, rc_v, w_v, y_v, acc_v, bin_v,
             psum_v, pprt_v, zb_v, idx80_v, iidx_v, x_v, win_v, oidx_v,
             wout_v, oval_v, sh_sum, sh_y, xsem, dsem, hsem):
    cid = lax.axis_index("c")
    sid = lax.axis_index("s")
    eid = cid * NS + sid
    lo = sid * SHARE
    elo = eid * EPT

    pltpu.sync_copy(rc_hbm.at[pl.ds(elo, EPT)], rc_v)
    pltpu.sync_copy(w_hbm.at[pl.ds(elo, EPT)], w_v)
    pltpu.sync_copy(bias_hbm.at[pl.ds(lo, SHARE)], bin_v)
    pltpu.sync_copy(iidx_hbm, iidx_v)
    pltpu.sync_copy(x_hbm, x_v)
    pltpu.sync_copy(win_hbm, win_v)

    def inproj(g, c):
        ii = iidx_v[pl.ds(g * L, L)]
        v = x_v[pl.ds(g * L, L)] * win_v[pl.ds(g * L, L)]
        m = (ii >= lo) & (ii < lo + SHARE)
        iloc = jnp.where(m, ii - lo, 0)
        plsc.addupdate_scatter(bin_v, [iloc], jnp.where(m, v, 0.0), mask=m)
        return c

    lax.fori_loop(0, N_IN // L, inproj, 0)

    zeros = jnp.zeros((L,), jnp.float32)
    lanes = lax.iota(jnp.int32, L)

    def init_bufs(i, c):
        y_v[pl.ds(i * L, L)] = zeros
        r = i >> 3
        acc_v[r, pl.ds((i & 7) * L, L)] = zeros
        return c

    lax.fori_loop(0, NPAD // L, init_bufs, 0)

    def init_zb(i, c):
        zb_v[i >> 3, pl.ds((i & 7) * L, L)] = zeros
        return c

    lax.fori_loop(0, RPS * 8, init_zb, 0)

    def init_idx(g, c):
        idx80_v[pl.ds(g * L, L)] = lanes + g * L
        return c

    lax.fori_loop(0, ROWS128 // L, init_idx, 0)

    pltpu.sync_copy(zb_v, sh_sum.at[pl.ds(sid * RPS, RPS)])

    def iteration(it, c):
        @plsc.parallel_loop(0, GROUPS, unroll=UNROLL)
        def egrp(g):
            base = g * L
            rc = rc_v[pl.ds(base, L)]
            col = rc & COLMASK
            row = rc >> COLBITS
            yv = plsc.load_gather(y_v, [col])
            wv = w_v[pl.ds(base, L)]
            plsc.addupdate_scatter(acc_v, [row >> 7, row & 127], yv * wv)

        pltpu.sync_copy(acc_v, sh_sum.at[idx80_v], add=True)
        plsc.subcore_barrier()
        pltpu.sync_copy(sh_sum.at[pl.ds(sid * RPS, RPS)], psum_v)

        par = lax.rem(it, 2)
        pltpu.sync_copy(psum_v, part_hbm.at[par, cid, sid])
        pltpu.semaphore_signal(xsem, 1, core_index=1 - cid)
        pltpu.sync_copy(zb_v, sh_sum.at[pl.ds(sid * RPS, RPS)])
        pl.semaphore_wait(xsem, 1)
        pltpu.sync_copy(part_hbm.at[par, 1 - cid, sid], pprt_v)

        HALF = SHARE // 2

        @plsc.parallel_loop(0, SHARE // L // 2, unroll=4)
        def red2a(g):
            r = g >> 3
            cc = (g & 7) * L
            z = (psum_v[r, pl.ds(cc, L)] + pprt_v[r, pl.ds(cc, L)]
                 + bin_v[pl.ds(g * L, L)])
            y_v[pl.ds(lo + g * L, L)] = _act(z)

        dpub = pltpu.async_copy(y_v.at[pl.ds(lo, HALF)],
                                sh_y.at[pl.ds(lo, HALF)], hsem)

        @plsc.parallel_loop(SHARE // L // 2, SHARE // L, unroll=4)
        def red2b(g):
            r = g >> 3
            cc = (g & 7) * L
            z = (psum_v[r, pl.ds(cc, L)] + pprt_v[r, pl.ds(cc, L)]
                 + bin_v[pl.ds(g * L, L)])
            y_v[pl.ds(lo + g * L, L)] = _act(z)

        pltpu.sync_copy(y_v.at[pl.ds(lo + HALF, HALF)],
                        sh_y.at[pl.ds(lo + HALF, HALF)])
        dpub.wait()
        plsc.subcore_barrier()
        desc = pltpu.async_copy(sh_y, y_v, dsem)

        @plsc.parallel_loop(0, ROWS128, unroll=8)
        def zero_acc(r):
            for u in range(8):
                acc_v[r, pl.ds(u * L, L)] = zeros

        desc.wait()
        return c

    lax.fori_loop(0, ITERATIONS, iteration, 0)

    @pl.when(cid == 0)
    def _():
        pltpu.sync_copy(oidx_hbm.at[pl.ds(sid * L, L)], oidx_v)
        pltpu.sync_copy(wout_hbm.at[pl.ds(sid * L, L)], wout_v)
        oi = oidx_v[pl.ds(0, L)]
        ov = plsc.load_gather(y_v, [oi]) * wout_v[pl.ds(0, L)]
        oval_v[pl.ds(0, L)] = ov
        pltpu.sync_copy(oval_v, out_hbm.at[pl.ds(sid * L, L)])


_sc_kernel = functools.partial(
    pl.kernel,
    out_type=(jax.ShapeDtypeStruct((N_OUT,), jnp.float32),
              jax.ShapeDtypeStruct((2, NC, NS, RPS, 128), jnp.float32)),
    mesh=plsc.VectorSubcoreMesh(
        core_axis_name="c", subcore_axis_name="s", num_cores=NC),
    compiler_params=pltpu.CompilerParams(needs_layout_passes=False),
    scratch_types=[
        pltpu.VMEM((EPT,), jnp.int32),
        pltpu.VMEM((EPT,), jnp.float32),
        pltpu.VMEM((NPAD,), jnp.float32),
        pltpu.VMEM((ROWS128, 128), jnp.float32),
        pltpu.VMEM((SHARE,), jnp.float32),
        pltpu.VMEM((RPS, 128), jnp.float32),
        pltpu.VMEM((RPS, 128), jnp.float32),
        pltpu.VMEM((RPS, 128), jnp.float32),
        pltpu.VMEM((ROWS128,), jnp.int32),
        pltpu.VMEM((N_IN,), jnp.int32),
        pltpu.VMEM((N_IN,), jnp.float32),
        pltpu.VMEM((N_IN,), jnp.float32),
        pltpu.VMEM((L,), jnp.int32),
        pltpu.VMEM((L,), jnp.float32),
        pltpu.VMEM((L,), jnp.float32),
        pltpu.VMEM_SHARED((ROWS128, 128), jnp.float32),
        pltpu.VMEM_SHARED((NPAD,), jnp.float32),
        pltpu.SemaphoreType.REGULAR,
        pltpu.SemaphoreType.DMA,
        pltpu.SemaphoreType.DMA,
    ],
)(_sc_body)


def kernel(x, w_in, w_rec, bias, w_out, rows, cols, in_idx, out_idx):
    rc = rows.astype(jnp.int32) * (1 << COLBITS) + cols.astype(jnp.int32)
    bias_pad = jnp.pad(bias.reshape(-1), (0, NPAD - N_NODES))
    out, _ = _sc_kernel(rc, w_rec, bias_pad, x.reshape(-1), w_in,
                        in_idx, out_idx, w_out)
    return out.reshape(1, N_OUT)
